# baseline (speedup 1.0000x reference)
import functools

import jax
import jax.numpy as jnp
from jax import lax
from jax.experimental import pallas as pl
from jax.experimental.pallas import tpu as pltpu
from jax.experimental.pallas import tpu_sc as plsc

NW = 32
BT = 128
NBUF = 4


def _gather_kernel(per_w, H, x_hbm, table_hbm, out_hbm,
                   idx_v, idx_t, rows_v, trans_v, gsems, osems):
    nc = 2
    wid = lax.axis_index("s") * nc + lax.axis_index("c")
    s0 = wid * per_w
    ntile = per_w // BT
    nblk = ntile * H
    lane = lax.iota(jnp.int32, 16)

    pltpu.sync_copy(x_hbm.at[pl.ds(s0, per_w)], idx_v)

    @pl.loop(0, H)
    def _(h):
        cols = lane * 0 + h
        for chunk in range(per_w // 16):
            v = plsc.load_gather(idx_v, [chunk * 16 + lane, cols])
            idx_t[h, pl.ds(chunk * 16, 16)] = v

    @pl.loop(0, nblk, step=NBUF)
    def group(i0):
        descs = []
        for b in range(NBUF):
            i = i0 + b
            h = lax.rem(i, H)
            bsub = lax.div(i, H)
            @pl.when(i0 > 0)
            def _(b=b):
                pltpu.make_async_copy(
                    trans_v.at[b], out_hbm.at[0, :, 0], osems[b]
                ).wait()
            descs.append(
                pltpu.async_copy(
                    table_hbm.at[idx_t.at[h, pl.ds(bsub * BT, BT)]],
                    rows_v.at[b], gsems[b],
                )
            )
        for b in range(NBUF):
            i = i0 + b
            h = lax.rem(i, H)
            bsub = lax.div(i, H)
            bj = wid * ntile + bsub
            descs[b].wait()

            @pl.loop(0, 64)
            def _(d, b=b):
                ti = lax.div(d, 8)
                di = lax.rem(d, 8)
                cols = lane * 0 + d
                for chunk in range(BT // 16):
                    v = plsc.load_gather(rows_v.at[b], [chunk * 16 + lane, cols])
                    trans_v[b, ti, di, pl.ds(chunk * 16, 16)] = v

            pltpu.async_copy(
                trans_v.at[b], out_hbm.at[h, :, bj], osems[b]
            )

    for b in range(NBUF):
        pltpu.make_async_copy(
            trans_v.at[b], out_hbm.at[0, :, 0], osems[b]
        ).wait()


def kernel(x, table):
    B, H = x.shape
    V, D = table.shape
    per_w = B // NW
    assert per_w * NW == B and per_w % BT == 0 and D == 64 and BT == 128

    mesh = plsc.VectorSubcoreMesh(core_axis_name="c", subcore_axis_name="s")
    run = pl.kernel(
        functools.partial(_gather_kernel, per_w, H),
        out_type=jax.ShapeDtypeStruct((H, D // 8, B // BT, 8, BT), jnp.float32),
        mesh=mesh,
        scratch_types=[
            pltpu.VMEM((per_w, H), jnp.int32),
            pltpu.VMEM((H, per_w), jnp.int32),
            pltpu.VMEM((NBUF, BT, D), jnp.float32),
            pltpu.VMEM((NBUF, D // 8, 8, BT), jnp.float32),
            [pltpu.SemaphoreType.DMA] * NBUF,
            [pltpu.SemaphoreType.DMA] * NBUF,
        ],
        compiler_params=pltpu.CompilerParams(use_tc_tiling_on_sc=False, needs_layout_passes=False),
    )
    out5 = run(x.astype(jnp.int32), table)
    return out5.transpose(2, 4, 0, 1, 3).reshape(B, H, D)
